# baseline (speedup 1.0000x reference)
import functools

import jax
import jax.numpy as jnp
from jax import lax
from jax.experimental import pallas as pl
from jax.experimental.pallas import tpu as pltpu

NUM_EMB = 1000
D = 64
BATCH = 16384
HIST = 200

B1 = 64
GRID = BATCH // B1


def _tc_body(idx_ref, emb_ref, out_ref):
    idxf = idx_ref[...].astype(jnp.float32)
    alpha = (999.0 - idxf) / 999.0
    e0 = emb_ref[0, :]
    e1 = emb_ref[1, :]
    d = e0 - e1
    out_ref[...] = alpha[:, :, None] * d[None, None, :] + e1[None, None, :]


def kernel(index_tensor, embedding_matrix, interpolation_matrix):
    return pl.pallas_call(
        _tc_body,
        grid=(GRID,),
        in_specs=[pl.BlockSpec((B1, HIST), lambda i: (i, 0)),
                  pl.BlockSpec((2, D), lambda i: (0, 0))],
        out_specs=pl.BlockSpec((B1, HIST, D), lambda i: (i, 0, 0)),
        out_shape=jax.ShapeDtypeStruct((BATCH, HIST, D), jnp.float32),
    )(index_tensor.astype(jnp.int32), embedding_matrix)
